# baseline (device time: 94210 ns/iter reference)
import jax
import jax.numpy as jnp
from jax import lax
from jax.experimental import pallas as pl
from jax.experimental.pallas import tpu as pltpu

N_DEV = 8
N_HOP = N_DEV - 1
N_SUB = 4


def kernel(x, w_mat, scale_x, scale_w):
    m_per, k = x.shape
    half = m_per // 2
    qtr = half // N_SUB
    _, n_per = w_mat.shape

    x8 = x.astype(jnp.float8_e4m3fn)
    w16 = w_mat.astype(jnp.bfloat16)
    s = (scale_x * scale_w).reshape(1, 1)

    def body(s_ref, x_ref, w_ref, out_ref,
             cw_ref, ccw_ref, send_cw, recv_cw, send_ccw, recv_ccw):
        my = lax.axis_index("i")
        left = lax.rem(my + N_DEV - 1, N_DEV)
        right = lax.rem(my + 1, N_DEV)

        barrier_sem = pltpu.get_barrier_semaphore()
        for nbr in (left, right):
            pl.semaphore_signal(
                barrier_sem, inc=1,
                device_id=(nbr,), device_id_type=pl.DeviceIdType.MESH,
            )
        pl.semaphore_wait(barrier_sem, 2)

        def hop_rdmas(h, q):
            row = pl.ds(q * qtr, qtr)
            cw = pltpu.make_async_remote_copy(
                src_ref=(x_ref.at[pl.ds(q * qtr, qtr)] if h == 0
                         else cw_ref.at[h - 1, row]),
                dst_ref=cw_ref.at[h, row],
                send_sem=send_cw.at[h, q],
                recv_sem=recv_cw.at[h, q],
                device_id=(right,),
                device_id_type=pl.DeviceIdType.MESH,
            )
            ccw = pltpu.make_async_remote_copy(
                src_ref=(x_ref.at[pl.ds(half + q * qtr, qtr)] if h == 0
                         else ccw_ref.at[h - 1, row]),
                dst_ref=ccw_ref.at[h, row],
                send_sem=send_ccw.at[h, q],
                recv_sem=recv_ccw.at[h, q],
                device_id=(left,),
                device_id_type=pl.DeviceIdType.MESH,
            )
            return cw, ccw

        sc = s_ref[0, 0]
        w = w_ref[...]

        rdmas = {(0, q): hop_rdmas(0, q) for q in range(N_SUB)}
        for q in range(N_SUB):
            rdmas[0, q][0].start()
            rdmas[0, q][1].start()

        out_ref[pl.ds(my * m_per, m_per), :] = (
            jnp.dot(x_ref[...].astype(jnp.bfloat16), w,
                    preferred_element_type=jnp.float32) * sc
        )

        for h in range(N_HOP):
            for q in range(N_SUB):
                cw, ccw = rdmas[h, q]
                cw.wait_recv()
                ccw.wait_recv()
                if h + 1 < N_HOP:
                    nxt = hop_rdmas(h + 1, q)
                    nxt[0].start()
                    nxt[1].start()
                    rdmas[h + 1, q] = nxt
            o_cw = lax.rem(my + N_DEV - 1 - h, N_DEV)
            o_ccw = lax.rem(my + 1 + h, N_DEV)
            out_ref[pl.ds(o_cw * m_per, half), :] = (
                jnp.dot(cw_ref[h].astype(jnp.bfloat16), w,
                        preferred_element_type=jnp.float32) * sc
            )
            out_ref[pl.ds(o_ccw * m_per + half, half), :] = (
                jnp.dot(ccw_ref[h].astype(jnp.bfloat16), w,
                        preferred_element_type=jnp.float32) * sc
            )

        for cw, ccw in rdmas.values():
            cw.wait_send()
            ccw.wait_send()

    return pl.pallas_call(
        body,
        out_shape=jax.ShapeDtypeStruct((N_DEV * m_per, n_per), jnp.float32),
        in_specs=[
            pl.BlockSpec(memory_space=pltpu.SMEM),
            pl.BlockSpec(memory_space=pltpu.VMEM),
            pl.BlockSpec(memory_space=pltpu.VMEM),
        ],
        out_specs=pl.BlockSpec(memory_space=pltpu.VMEM),
        scratch_shapes=[
            pltpu.VMEM((N_HOP, half, k), jnp.float8_e4m3fn),
            pltpu.VMEM((N_HOP, half, k), jnp.float8_e4m3fn),
            pltpu.SemaphoreType.DMA((N_HOP, N_SUB)),
            pltpu.SemaphoreType.DMA((N_HOP, N_SUB)),
            pltpu.SemaphoreType.DMA((N_HOP, N_SUB)),
            pltpu.SemaphoreType.DMA((N_HOP, N_SUB)),
        ],
        compiler_params=pltpu.CompilerParams(collective_id=0),
    )(s, x8, w16)


# device time: 86821 ns/iter; 1.0851x vs baseline; 1.0851x over previous
import jax
import jax.numpy as jnp
from jax import lax
from jax.experimental import pallas as pl
from jax.experimental.pallas import tpu as pltpu

N_DEV = 8


def kernel(x, w_mat, scale_x, scale_w):
    m_per, k = x.shape
    _, n_per = w_mat.shape
    kh = k // 2
    kq = k // 4

    x8 = x.astype(jnp.float8_e4m3fn)
    w8 = w_mat.astype(jnp.float8_e5m2)
    s = (scale_x * scale_w).reshape(1, 1)

    def body(s_ref, x_ref, w_ref, out_ref,
             x16_ref, wcw_ref, wccw_ref, res_ref,
             send_cw, recv_cw, send_ccw, recv_ccw, res_send, res_recv):
        my = lax.axis_index("i")
        left = lax.rem(my + N_DEV - 1, N_DEV)
        right = lax.rem(my + 1, N_DEV)

        barrier_sem = pltpu.get_barrier_semaphore()
        for off in range(1, N_DEV):
            pl.semaphore_signal(
                barrier_sem, inc=1,
                device_id=(lax.rem(my + off, N_DEV),),
                device_id_type=pl.DeviceIdType.MESH,
            )
        pl.semaphore_wait(barrier_sem, N_DEV - 1)

        def ring_rdma(dirn, h, sub):
            buf = wcw_ref if dirn == 0 else wccw_ref
            if h < 3:
                rows = pl.ds(sub * kh, kh)
                src = w_ref.at[rows] if h == 0 else buf.at[h - 1, rows]
            else:
                base = 0 if dirn == 0 else kh
                rows = pl.ds(base + sub * kq, kq)
                src = buf.at[2, rows]
            return pltpu.make_async_remote_copy(
                src_ref=src,
                dst_ref=buf.at[h, rows],
                send_sem=(send_cw if dirn == 0 else send_ccw).at[h, sub],
                recv_sem=(recv_cw if dirn == 0 else recv_ccw).at[h, sub],
                device_id=(right if dirn == 0 else left,),
                device_id_type=pl.DeviceIdType.MESH,
            )

        def res_rdma(slot, dst_dev):
            return pltpu.make_async_remote_copy(
                src_ref=res_ref.at[slot],
                dst_ref=out_ref.at[pl.ds(my * m_per, m_per)],
                send_sem=res_send.at[slot],
                recv_sem=res_recv.at[slot],
                device_id=(dst_dev,),
                device_id_type=pl.DeviceIdType.MESH,
            )

        sc = s_ref[0, 0]
        rdmas = {}
        for dirn in range(2):
            for sub in range(2):
                rdmas[dirn, 0, sub] = ring_rdma(dirn, 0, sub)
                rdmas[dirn, 0, sub].start()

        x16_ref[...] = x_ref[...].astype(jnp.bfloat16)
        out_ref[pl.ds(my * m_per, m_per), :] = (
            jnp.dot(x16_ref[...], w_ref[...].astype(jnp.bfloat16),
                    preferred_element_type=jnp.float32) * sc
        )

        res_sends = []
        for h in range(3):
            for sub in range(2):
                rdmas[0, h, sub].wait_recv()
                if h < 2:
                    rdmas[0, h + 1, sub] = ring_rdma(0, h + 1, sub)
                    rdmas[0, h + 1, sub].start()
                elif sub == 0:
                    for s3 in range(2):
                        rdmas[0, 3, s3] = ring_rdma(0, 3, s3)
                        rdmas[0, 3, s3].start()
                rdmas[1, h, sub].wait_recv()
                if h < 2:
                    rdmas[1, h + 1, sub] = ring_rdma(1, h + 1, sub)
                    rdmas[1, h + 1, sub].start()
                elif sub == 1:
                    for s3 in range(2):
                        rdmas[1, 3, s3] = ring_rdma(1, 3, s3)
                        rdmas[1, 3, s3].start()
            o_cw = lax.rem(my + N_DEV - 1 - h, N_DEV)
            res_ref[h, :, :] = (
                jnp.dot(x16_ref[...], wcw_ref[h].astype(jnp.bfloat16),
                        preferred_element_type=jnp.float32) * sc
            )
            r = res_rdma(h, o_cw)
            r.start()
            res_sends.append(r)
            o_ccw = lax.rem(my + 1 + h, N_DEV)
            res_ref[6 - h, :, :] = (
                jnp.dot(x16_ref[...], wccw_ref[h].astype(jnp.bfloat16),
                        preferred_element_type=jnp.float32) * sc
            )
            r = res_rdma(6 - h, o_ccw)
            r.start()
            res_sends.append(r)

        for sub in range(2):
            rdmas[0, 3, sub].wait_recv()
            rdmas[1, 3, sub].wait_recv()
        o4 = lax.rem(my + 4, N_DEV)
        res_ref[3, :, :] = (
            (jnp.dot(x16_ref[:, :kh], wcw_ref[3, :kh].astype(jnp.bfloat16),
                     preferred_element_type=jnp.float32)
             + jnp.dot(x16_ref[:, kh:], wccw_ref[3, kh:].astype(jnp.bfloat16),
                       preferred_element_type=jnp.float32)) * sc
        )
        r = res_rdma(3, o4)
        r.start()
        res_sends.append(r)

        for slot in range(N_DEV - 1):
            d = lax.rem(my + slot + 1, N_DEV)
            recv = pltpu.make_async_remote_copy(
                src_ref=res_ref.at[slot],
                dst_ref=out_ref.at[pl.ds(d * m_per, m_per)],
                send_sem=res_send.at[slot],
                recv_sem=res_recv.at[slot],
                device_id=(d,),
                device_id_type=pl.DeviceIdType.MESH,
            )
            recv.wait_recv()

        for r in rdmas.values():
            r.wait_send()
        for r in res_sends:
            r.wait_send()

    return pl.pallas_call(
        body,
        out_shape=jax.ShapeDtypeStruct((N_DEV * m_per, n_per), jnp.float32),
        in_specs=[
            pl.BlockSpec(memory_space=pltpu.SMEM),
            pl.BlockSpec(memory_space=pltpu.VMEM),
            pl.BlockSpec(memory_space=pltpu.VMEM),
        ],
        out_specs=pl.BlockSpec(memory_space=pltpu.VMEM),
        scratch_shapes=[
            pltpu.VMEM((m_per, k), jnp.bfloat16),
            pltpu.VMEM((4, k, n_per), jnp.float8_e5m2),
            pltpu.VMEM((4, k, n_per), jnp.float8_e5m2),
            pltpu.VMEM((N_DEV - 1, m_per, n_per), jnp.float32),
            pltpu.SemaphoreType.DMA((4, 2)),
            pltpu.SemaphoreType.DMA((4, 2)),
            pltpu.SemaphoreType.DMA((4, 2)),
            pltpu.SemaphoreType.DMA((4, 2)),
            pltpu.SemaphoreType.DMA((N_DEV - 1,)),
            pltpu.SemaphoreType.DMA((N_DEV - 1,)),
        ],
        compiler_params=pltpu.CompilerParams(collective_id=0),
    )(s, x8, w8)


# device time: 69972 ns/iter; 1.3464x vs baseline; 1.2408x over previous
import jax
import jax.numpy as jnp
from jax import lax
from jax.experimental import pallas as pl
from jax.experimental.pallas import tpu as pltpu

N_DEV = 8


def kernel(x, w_mat, scale_x, scale_w):
    m_per, k = x.shape
    _, n_per = w_mat.shape
    kh = k // 2
    kq = k // 4

    x8 = x.astype(jnp.float8_e4m3fn)
    w8 = w_mat.astype(jnp.float8_e5m2)
    s = (scale_x * scale_w).reshape(1, 1)

    def body(s_ref, x_ref, w_ref, out_ref,
             x16_ref, wcw_ref, wccw_ref, res_ref, rcv_ref,
             send_cw, recv_cw, send_ccw, recv_ccw, res_send, res_recv):
        my = lax.axis_index("i")
        my_pos = jnp.where(my < 4, my, 11 - my)

        def dev_at(pos_off):
            p = lax.rem(my_pos + pos_off + 2 * N_DEV, N_DEV)
            return jnp.where(p < 4, p, 11 - p)

        left = dev_at(-1)
        right = dev_at(1)

        barrier_sem = pltpu.get_barrier_semaphore()
        for off in range(1, N_DEV):
            pl.semaphore_signal(
                barrier_sem, inc=1,
                device_id=(lax.rem(my + off, N_DEV),),
                device_id_type=pl.DeviceIdType.MESH,
            )
        pl.semaphore_wait(barrier_sem, N_DEV - 1)

        def ring_rdma(dirn, h, sub):
            buf = wcw_ref if dirn == 0 else wccw_ref
            if h < 3:
                rows = pl.ds(sub * kh, kh)
                src = w_ref.at[rows] if h == 0 else buf.at[h - 1, rows]
            else:
                base = 0 if dirn == 0 else kh
                rows = pl.ds(base + sub * kq, kq)
                src = buf.at[2, rows]
            return pltpu.make_async_remote_copy(
                src_ref=src,
                dst_ref=buf.at[h, rows],
                send_sem=(send_cw if dirn == 0 else send_ccw).at[h, sub],
                recv_sem=(recv_cw if dirn == 0 else recv_ccw).at[h, sub],
                device_id=(right if dirn == 0 else left,),
                device_id_type=pl.DeviceIdType.MESH,
            )

        def res_rdma(slot, dst_dev):
            return pltpu.make_async_remote_copy(
                src_ref=res_ref.at[slot],
                dst_ref=rcv_ref.at[slot],
                send_sem=res_send.at[slot],
                recv_sem=res_recv.at[slot],
                device_id=(dst_dev,),
                device_id_type=pl.DeviceIdType.MESH,
            )

        sc = s_ref[0, 0]
        rdmas = {}
        for dirn in range(2):
            for sub in range(2):
                rdmas[dirn, 0, sub] = ring_rdma(dirn, 0, sub)
                rdmas[dirn, 0, sub].start()

        x16_ref[...] = x_ref[...].astype(jnp.bfloat16)
        out_ref[pl.ds(my * m_per, m_per), :] = (
            jnp.dot(x16_ref[...], w_ref[...].astype(jnp.bfloat16),
                    preferred_element_type=jnp.float32) * sc
        )

        res_sends = []
        for h in range(3):
            for sub in range(2):
                rdmas[0, h, sub].wait_recv()
                if h < 2:
                    rdmas[0, h + 1, sub] = ring_rdma(0, h + 1, sub)
                    rdmas[0, h + 1, sub].start()
                elif sub == 0:
                    for s3 in range(2):
                        rdmas[0, 3, s3] = ring_rdma(0, 3, s3)
                        rdmas[0, 3, s3].start()
                rdmas[1, h, sub].wait_recv()
                if h < 2:
                    rdmas[1, h + 1, sub] = ring_rdma(1, h + 1, sub)
                    rdmas[1, h + 1, sub].start()
                elif sub == 1:
                    for s3 in range(2):
                        rdmas[1, 3, s3] = ring_rdma(1, 3, s3)
                        rdmas[1, 3, s3].start()
            o_cw = dev_at(-(h + 1))
            res_ref[h, :, :] = (
                jnp.dot(x16_ref[...], wcw_ref[h].astype(jnp.bfloat16),
                        preferred_element_type=jnp.float32) * sc
            ).astype(jnp.bfloat16)
            r = res_rdma(h, o_cw)
            r.start()
            res_sends.append(r)
            o_ccw = dev_at(h + 1)
            res_ref[6 - h, :, :] = (
                jnp.dot(x16_ref[...], wccw_ref[h].astype(jnp.bfloat16),
                        preferred_element_type=jnp.float32) * sc
            ).astype(jnp.bfloat16)
            r = res_rdma(6 - h, o_ccw)
            r.start()
            res_sends.append(r)

        for sub in range(2):
            rdmas[0, 3, sub].wait_recv()
            rdmas[1, 3, sub].wait_recv()
        o4 = dev_at(4)
        res_ref[3, :, :] = (
            (jnp.dot(x16_ref[:, :kh], wcw_ref[3, :kh].astype(jnp.bfloat16),
                     preferred_element_type=jnp.float32)
             + jnp.dot(x16_ref[:, kh:], wccw_ref[3, kh:].astype(jnp.bfloat16),
                       preferred_element_type=jnp.float32)) * sc
        ).astype(jnp.bfloat16)
        r = res_rdma(3, o4)
        r.start()
        res_sends.append(r)

        for slot in range(N_DEV - 1):
            d = dev_at(slot + 1)
            recv = pltpu.make_async_remote_copy(
                src_ref=res_ref.at[slot],
                dst_ref=rcv_ref.at[slot],
                send_sem=res_send.at[slot],
                recv_sem=res_recv.at[slot],
                device_id=(d,),
                device_id_type=pl.DeviceIdType.MESH,
            )
            recv.wait_recv()
            out_ref[pl.ds(d * m_per, m_per), :] = rcv_ref[slot].astype(
                jnp.float32)

        for r in rdmas.values():
            r.wait_send()
        for r in res_sends:
            r.wait_send()

    return pl.pallas_call(
        body,
        out_shape=jax.ShapeDtypeStruct((N_DEV * m_per, n_per), jnp.float32),
        in_specs=[
            pl.BlockSpec(memory_space=pltpu.SMEM),
            pl.BlockSpec(memory_space=pltpu.VMEM),
            pl.BlockSpec(memory_space=pltpu.VMEM),
        ],
        out_specs=pl.BlockSpec(memory_space=pltpu.VMEM),
        scratch_shapes=[
            pltpu.VMEM((m_per, k), jnp.bfloat16),
            pltpu.VMEM((4, k, n_per), jnp.float8_e5m2),
            pltpu.VMEM((4, k, n_per), jnp.float8_e5m2),
            pltpu.VMEM((N_DEV - 1, m_per, n_per), jnp.bfloat16),
            pltpu.VMEM((N_DEV - 1, m_per, n_per), jnp.bfloat16),
            pltpu.SemaphoreType.DMA((4, 2)),
            pltpu.SemaphoreType.DMA((4, 2)),
            pltpu.SemaphoreType.DMA((4, 2)),
            pltpu.SemaphoreType.DMA((4, 2)),
            pltpu.SemaphoreType.DMA((N_DEV - 1,)),
            pltpu.SemaphoreType.DMA((N_DEV - 1,)),
        ],
        compiler_params=pltpu.CompilerParams(collective_id=0),
    )(s, x8, w8)


# device time: 66393 ns/iter; 1.4190x vs baseline; 1.0539x over previous
import jax
import jax.numpy as jnp
from jax import lax
from jax.experimental import pallas as pl
from jax.experimental.pallas import tpu as pltpu

N_DEV = 8


def kernel(x, w_mat, scale_x, scale_w):
    m_per, k = x.shape
    _, n_per = w_mat.shape
    kh = k // 2
    kq = k // 4

    x8 = x.astype(jnp.float8_e4m3fn)
    w8 = w_mat.astype(jnp.float8_e5m2)
    s = (scale_x * scale_w).reshape(1, 1)

    def body(s_ref, x_ref, w_ref, out_ref,
             x16_ref, mate_ref, cw_ref, ccw_ref, res_ref, rcv_ref,
             z_send, z_recv, send_cw, recv_cw, send_ccw, recv_ccw,
             res_send, res_recv):
        my = lax.axis_index("i")
        plb = my // 4 * 4
        q = lax.rem(my, 4)
        crossb = 4 - plb

        def same_at(qq):
            return plb + lax.rem(qq + 8, 4)

        def cross_at(qq):
            return crossb + lax.rem(qq + 8, 4)

        left = same_at(q - 1)
        right = same_at(q + 1)
        mate = crossb + q

        barrier_sem = pltpu.get_barrier_semaphore()
        for off in range(1, N_DEV):
            pl.semaphore_signal(
                barrier_sem, inc=1,
                device_id=(lax.rem(my + off, N_DEV),),
                device_id_type=pl.DeviceIdType.MESH,
            )
        pl.semaphore_wait(barrier_sem, N_DEV - 1)

        Z_BASES = (0, kh, kq, kh + kq)
        z_rdmas = []
        for i, base in enumerate(Z_BASES):
            r = pltpu.make_async_remote_copy(
                src_ref=w_ref.at[pl.ds(base, kq)],
                dst_ref=mate_ref.at[pl.ds(base, kq)],
                send_sem=z_send.at[i],
                recv_sem=z_recv.at[i],
                device_id=(mate,),
                device_id_type=pl.DeviceIdType.MESH,
            )
            r.start()
            z_rdmas.append(r)

        def ring_rdma(dirn, h, j, sub):
            buf = cw_ref if dirn == 0 else ccw_ref
            rows = pl.ds(sub * kq, kq)
            if h == 0:
                src_full = w_ref if j == 0 else mate_ref
                base = (0 if dirn == 0 else kh) + sub * kq
                src = src_full.at[pl.ds(base, kq)]
            else:
                src = buf.at[h - 1, j, rows]
            return pltpu.make_async_remote_copy(
                src_ref=src,
                dst_ref=buf.at[h, j, rows],
                send_sem=(send_cw if dirn == 0 else send_ccw).at[h, j, sub],
                recv_sem=(recv_cw if dirn == 0 else recv_ccw).at[h, j, sub],
                device_id=(right if dirn == 0 else left,),
                device_id_type=pl.DeviceIdType.MESH,
            )

        def res_rdma(slot, dst_dev):
            return pltpu.make_async_remote_copy(
                src_ref=res_ref.at[slot],
                dst_ref=rcv_ref.at[slot],
                send_sem=res_send.at[slot],
                recv_sem=res_recv.at[slot],
                device_id=(dst_dev,),
                device_id_type=pl.DeviceIdType.MESH,
            )

        sc = s_ref[0, 0]
        rdmas = {}
        for dirn in range(2):
            for sub in range(2):
                rdmas[dirn, 0, 0, sub] = ring_rdma(dirn, 0, 0, sub)
                rdmas[dirn, 0, 0, sub].start()

        x16_ref[...] = x_ref[...].astype(jnp.bfloat16)
        out_ref[pl.ds(my * m_per, m_per), :] = (
            jnp.dot(x16_ref[...], w_ref[...].astype(jnp.bfloat16),
                    preferred_element_type=jnp.float32) * sc
        )

        res_sends = []

        def gemm_block(slot, dst_dev, top, bot):
            res_ref[slot, :, :] = (
                (jnp.dot(x16_ref[:, :kh], top.astype(jnp.bfloat16),
                         preferred_element_type=jnp.float32)
                 + jnp.dot(x16_ref[:, kh:], bot.astype(jnp.bfloat16),
                           preferred_element_type=jnp.float32)) * sc
            ).astype(jnp.bfloat16)
            r = res_rdma(slot, dst_dev)
            r.start()
            res_sends.append(r)

        z_rdmas[0].wait_recv()
        rdmas[0, 0, 1, 0] = ring_rdma(0, 0, 1, 0)
        rdmas[0, 0, 1, 0].start()
        z_rdmas[1].wait_recv()
        rdmas[1, 0, 1, 0] = ring_rdma(1, 0, 1, 0)
        rdmas[1, 0, 1, 0].start()
        for dirn in range(2):
            rdmas[dirn, 0, 0, 0].wait_recv()
            rdmas[dirn, 1, 0, 0] = ring_rdma(dirn, 1, 0, 0)
            rdmas[dirn, 1, 0, 0].start()
        z_rdmas[2].wait_recv()
        rdmas[0, 0, 1, 1] = ring_rdma(0, 0, 1, 1)
        rdmas[0, 0, 1, 1].start()
        z_rdmas[3].wait_recv()
        rdmas[1, 0, 1, 1] = ring_rdma(1, 0, 1, 1)
        rdmas[1, 0, 1, 1].start()
        for dirn in range(2):
            rdmas[dirn, 0, 0, 1].wait_recv()
            rdmas[dirn, 1, 0, 1] = ring_rdma(dirn, 1, 0, 1)
            rdmas[dirn, 1, 0, 1].start()

        gemm_block(6, mate, mate_ref[:kh], mate_ref[kh:])

        for h in range(3):
            for j in ((1,) if h == 0 else (0, 1)):
                for sub in range(2):
                    for dirn in range(2):
                        rdmas[dirn, h, j, sub].wait_recv()
                        if h < 2:
                            nxt = ring_rdma(dirn, h + 1, j, sub)
                            nxt.start()
                            rdmas[dirn, h + 1, j, sub] = nxt
            if h == 1:
                gemm_block(1, same_at(q - 2), cw_ref[1, 0], ccw_ref[1, 0])
                gemm_block(4, cross_at(q - 2), cw_ref[1, 1], ccw_ref[1, 1])
            if h == 2:
                gemm_block(0, same_at(q - 1), cw_ref[0, 0], ccw_ref[2, 0])
                gemm_block(3, cross_at(q - 1), cw_ref[0, 1], ccw_ref[2, 1])
                gemm_block(2, same_at(q - 3), cw_ref[2, 0], ccw_ref[0, 0])
                gemm_block(5, cross_at(q - 3), cw_ref[2, 1], ccw_ref[0, 1])

        for slot in range(N_DEV - 1):
            if slot < 3:
                d = same_at(q + slot + 1)
            elif slot < 6:
                d = cross_at(q + slot - 2)
            else:
                d = mate
            recv = pltpu.make_async_remote_copy(
                src_ref=res_ref.at[slot],
                dst_ref=rcv_ref.at[slot],
                send_sem=res_send.at[slot],
                recv_sem=res_recv.at[slot],
                device_id=(d,),
                device_id_type=pl.DeviceIdType.MESH,
            )
            recv.wait_recv()
            out_ref[pl.ds(d * m_per, m_per), :] = rcv_ref[slot].astype(
                jnp.float32)

        for r in z_rdmas:
            r.wait_send()
        for r in rdmas.values():
            r.wait_send()
        for r in res_sends:
            r.wait_send()

    return pl.pallas_call(
        body,
        out_shape=jax.ShapeDtypeStruct((N_DEV * m_per, n_per), jnp.float32),
        in_specs=[
            pl.BlockSpec(memory_space=pltpu.SMEM),
            pl.BlockSpec(memory_space=pltpu.VMEM),
            pl.BlockSpec(memory_space=pltpu.VMEM),
        ],
        out_specs=pl.BlockSpec(memory_space=pltpu.VMEM),
        scratch_shapes=[
            pltpu.VMEM((m_per, k), jnp.bfloat16),
            pltpu.VMEM((k, n_per), jnp.float8_e5m2),
            pltpu.VMEM((3, 2, kh, n_per), jnp.float8_e5m2),
            pltpu.VMEM((3, 2, kh, n_per), jnp.float8_e5m2),
            pltpu.VMEM((N_DEV - 1, m_per, n_per), jnp.bfloat16),
            pltpu.VMEM((N_DEV - 1, m_per, n_per), jnp.bfloat16),
            pltpu.SemaphoreType.DMA((4,)),
            pltpu.SemaphoreType.DMA((4,)),
            pltpu.SemaphoreType.DMA((3, 2, 2)),
            pltpu.SemaphoreType.DMA((3, 2, 2)),
            pltpu.SemaphoreType.DMA((3, 2, 2)),
            pltpu.SemaphoreType.DMA((3, 2, 2)),
            pltpu.SemaphoreType.DMA((N_DEV - 1,)),
            pltpu.SemaphoreType.DMA((N_DEV - 1,)),
        ],
        compiler_params=pltpu.CompilerParams(collective_id=0),
    )(s, x8, w8)
